# baseline (device time: 23554 ns/iter reference)
import jax
import jax.numpy as jnp
from jax import lax
from jax.experimental import pallas as pl
from jax.experimental.pallas import tpu as pltpu

N_DEV = 8
B, SQ, D_MODEL = 2, 128, 512
H_PER, DH = 4, 64
ROWS = B * SQ
ROWS_PER = ROWS // N_DEV


def kernel(x, Wq, K_ext, V_ext, Wo):
    xf = x.reshape(ROWS, D_MODEL)

    def body(x_ref, wq_ref, k_hbm, v_hbm, wo_ref, out_ref,
             k_scr, v_scr, partial_ref, comm_a, comm_b, gather_ref,
             local_sems, send_a, recv_a, send_b, recv_b):
        my_pos = lax.axis_index("i")

        barrier_sem = pltpu.get_barrier_semaphore()
        for o in range(1, N_DEV):
            t = lax.rem(my_pos + o, N_DEV)
            pl.semaphore_signal(barrier_sem, inc=1, device_id=(t,),
                                device_id_type=pl.DeviceIdType.MESH)

        h0 = my_pos * H_PER
        k_dma = pltpu.make_async_copy(
            k_hbm.at[:, :, pl.ds(h0, H_PER), :], k_scr, local_sems.at[0])
        v_dma = pltpu.make_async_copy(
            v_hbm.at[:, :, pl.ds(h0, H_PER), :], v_scr, local_sems.at[1])
        k_dma.start()
        v_dma.start()

        q_all = jnp.dot(x_ref[:, :], wq_ref[:, :],
                        preferred_element_type=jnp.float32)
        k_dma.wait()
        v_dma.wait()
        for b in range(B):
            ctx_parts = []
            for h in range(H_PER):
                q = q_all[b * SQ:(b + 1) * SQ, h * DH:(h + 1) * DH]
                k = k_scr[b, :, h, :]
                v = v_scr[b, :, h, :]
                s = lax.dot_general(
                    q, k, (((1,), (1,)), ((), ())),
                    preferred_element_type=jnp.float32) * 0.125
                m = jnp.max(s, axis=-1, keepdims=True)
                e = jnp.exp(s - m)
                w = e / jnp.sum(e, axis=-1, keepdims=True)
                ctx_parts.append(jnp.dot(w, v,
                                         preferred_element_type=jnp.float32))
            ctx_b = jnp.concatenate(ctx_parts, axis=1)
            p_b = jnp.dot(ctx_b, wo_ref[:, :],
                          preferred_element_type=jnp.float32)
            partial_ref[4 * b:4 * (b + 1)] = p_b.astype(jnp.bfloat16).reshape(
                4, ROWS_PER, D_MODEL)

        pl.semaphore_wait(barrier_sem, N_DEV - 1)

        own_a = pltpu.make_async_copy(
            partial_ref.at[my_pos], comm_a.at[0], local_sems.at[0])
        own_a.start()
        sends_a = []
        for o in range(1, N_DEV):
            t = lax.rem(my_pos + o, N_DEV)
            rdma = pltpu.make_async_remote_copy(
                src_ref=partial_ref.at[t],
                dst_ref=comm_a.at[o],
                send_sem=send_a.at[o],
                recv_sem=recv_a.at[o],
                device_id=(t,),
                device_id_type=pl.DeviceIdType.MESH,
            )
            rdma.start()
            sends_a.append(rdma)
        own_a.wait()
        for o in range(1, N_DEV):
            pltpu.make_async_remote_copy(
                src_ref=comm_a.at[o],
                dst_ref=comm_a.at[o],
                send_sem=send_a.at[o],
                recv_sem=recv_a.at[o],
                device_id=(my_pos,),
                device_id_type=pl.DeviceIdType.MESH,
            ).wait_recv()
        red = comm_a[0].astype(jnp.float32)
        for o in range(1, N_DEV):
            red = red + comm_a[o].astype(jnp.float32)
        comm_b[:, :] = red.astype(jnp.bfloat16)

        sends_b = []
        for o in range(1, N_DEV):
            t = lax.rem(my_pos + o, N_DEV)
            rdma = pltpu.make_async_remote_copy(
                src_ref=comm_b,
                dst_ref=gather_ref.at[my_pos],
                send_sem=send_b.at[o],
                recv_sem=recv_b.at[my_pos],
                device_id=(t,),
                device_id_type=pl.DeviceIdType.MESH,
            )
            rdma.start()
            sends_b.append(rdma)
        for p in range(N_DEV):
            @pl.when(p == my_pos)
            def _():
                gather_ref[p] = comm_b[:, :]
        for p in range(N_DEV):
            @pl.when(p != my_pos)
            def _():
                pltpu.make_async_remote_copy(
                    src_ref=comm_b,
                    dst_ref=gather_ref.at[p],
                    send_sem=send_b.at[p],
                    recv_sem=recv_b.at[p],
                    device_id=(p,),
                    device_id_type=pl.DeviceIdType.MESH,
                ).wait_recv()
        out_ref[:, :] = gather_ref[:, :, :].reshape(
            ROWS, D_MODEL).astype(jnp.float32)
        for rdma in sends_a + sends_b:
            rdma.wait_send()

    out = pl.pallas_call(
        body,
        out_shape=jax.ShapeDtypeStruct((ROWS, D_MODEL), jnp.float32),
        in_specs=[
            pl.BlockSpec(memory_space=pltpu.VMEM),
            pl.BlockSpec(memory_space=pltpu.VMEM),
            pl.BlockSpec(memory_space=pl.ANY),
            pl.BlockSpec(memory_space=pl.ANY),
            pl.BlockSpec(memory_space=pltpu.VMEM),
        ],
        out_specs=pl.BlockSpec(memory_space=pltpu.VMEM),
        scratch_shapes=[
            pltpu.VMEM((B, SQ, H_PER, DH), jnp.float32),
            pltpu.VMEM((B, SQ, H_PER, DH), jnp.float32),
            pltpu.VMEM((N_DEV, ROWS_PER, D_MODEL), jnp.bfloat16),
            pltpu.VMEM((N_DEV, ROWS_PER, D_MODEL), jnp.bfloat16),
            pltpu.VMEM((ROWS_PER, D_MODEL), jnp.bfloat16),
            pltpu.VMEM((N_DEV, ROWS_PER, D_MODEL), jnp.bfloat16),
            pltpu.SemaphoreType.DMA((2,)),
            pltpu.SemaphoreType.DMA((N_DEV,)),
            pltpu.SemaphoreType.DMA((N_DEV,)),
            pltpu.SemaphoreType.DMA((N_DEV,)),
            pltpu.SemaphoreType.DMA((N_DEV,)),
        ],
        compiler_params=pltpu.CompilerParams(collective_id=0),
    )(xf, Wq, K_ext, V_ext, Wo)
    return out.reshape(B, SQ, D_MODEL)


# device time: 15810 ns/iter; 1.4898x vs baseline; 1.4898x over previous
import jax
import jax.numpy as jnp
from jax import lax
from jax.experimental import pallas as pl
from jax.experimental.pallas import tpu as pltpu

N_DEV = 8
B, SQ, D_MODEL = 2, 128, 512
H_PER, DH = 4, 64
ROWS = B * SQ
ROWS_PER = ROWS // N_DEV


def kernel(x, Wq, K_ext, V_ext, Wo):
    my = lax.axis_index("i")
    Ks = lax.dynamic_slice_in_dim(K_ext, my * H_PER, H_PER, axis=2)
    Vs = lax.dynamic_slice_in_dim(V_ext, my * H_PER, H_PER, axis=2)
    Ks = jnp.transpose(Ks, (0, 2, 1, 3))
    Vs = jnp.transpose(Vs, (0, 2, 1, 3))
    xf = x.reshape(ROWS, D_MODEL)

    def body(x_ref, wq_ref, k_ref, v_ref, wo_ref, out_ref,
             partial_ref, comm_a, comm_b, gather_ref,
             local_sems, send_a, recv_a, send_b, recv_b):
        my_pos = lax.axis_index("i")

        barrier_sem = pltpu.get_barrier_semaphore()
        for o in range(1, N_DEV):
            t = lax.rem(my_pos + o, N_DEV)
            pl.semaphore_signal(barrier_sem, inc=1, device_id=(t,),
                                device_id_type=pl.DeviceIdType.MESH)

        q_all = jnp.dot(x_ref[:, :], wq_ref[:, :],
                        preferred_element_type=jnp.float32)
        for b in range(B):
            ctx_parts = []
            for h in range(H_PER):
                q = q_all[b * SQ:(b + 1) * SQ, h * DH:(h + 1) * DH]
                k = k_ref[b, h]
                v = v_ref[b, h]
                s = lax.dot_general(
                    q, k, (((1,), (1,)), ((), ())),
                    preferred_element_type=jnp.float32) * 0.125
                m = jnp.max(s, axis=-1, keepdims=True)
                e = jnp.exp(s - m)
                w = e / jnp.sum(e, axis=-1, keepdims=True)
                ctx_parts.append(jnp.dot(w, v,
                                         preferred_element_type=jnp.float32))
            ctx_b = jnp.concatenate(ctx_parts, axis=1)
            p_b = jnp.dot(ctx_b, wo_ref[:, :],
                          preferred_element_type=jnp.float32)
            partial_ref[4 * b:4 * (b + 1)] = p_b.astype(jnp.bfloat16).reshape(
                4, ROWS_PER, D_MODEL)

        pl.semaphore_wait(barrier_sem, N_DEV - 1)

        own_a = pltpu.make_async_copy(
            partial_ref.at[my_pos], comm_a.at[0], local_sems.at[0])
        own_a.start()
        sends_a = []
        for o in range(1, N_DEV):
            t = lax.rem(my_pos + o, N_DEV)
            rdma = pltpu.make_async_remote_copy(
                src_ref=partial_ref.at[t],
                dst_ref=comm_a.at[o],
                send_sem=send_a.at[o],
                recv_sem=recv_a.at[o],
                device_id=(t,),
                device_id_type=pl.DeviceIdType.MESH,
            )
            rdma.start()
            sends_a.append(rdma)
        own_a.wait()
        for o in range(1, N_DEV):
            pltpu.make_async_remote_copy(
                src_ref=comm_a.at[o],
                dst_ref=comm_a.at[o],
                send_sem=send_a.at[o],
                recv_sem=recv_a.at[o],
                device_id=(my_pos,),
                device_id_type=pl.DeviceIdType.MESH,
            ).wait_recv()
        red = comm_a[0].astype(jnp.float32)
        for o in range(1, N_DEV):
            red = red + comm_a[o].astype(jnp.float32)
        comm_b[:, :] = red.astype(jnp.bfloat16)

        sends_b = []
        for o in range(1, N_DEV):
            t = lax.rem(my_pos + o, N_DEV)
            rdma = pltpu.make_async_remote_copy(
                src_ref=comm_b,
                dst_ref=gather_ref.at[my_pos],
                send_sem=send_b.at[o],
                recv_sem=recv_b.at[my_pos],
                device_id=(t,),
                device_id_type=pl.DeviceIdType.MESH,
            )
            rdma.start()
            sends_b.append(rdma)
        for p in range(N_DEV):
            @pl.when(p == my_pos)
            def _():
                gather_ref[p] = comm_b[:, :]
        for p in range(N_DEV):
            @pl.when(p != my_pos)
            def _():
                pltpu.make_async_remote_copy(
                    src_ref=comm_b,
                    dst_ref=gather_ref.at[p],
                    send_sem=send_b.at[p],
                    recv_sem=recv_b.at[p],
                    device_id=(p,),
                    device_id_type=pl.DeviceIdType.MESH,
                ).wait_recv()
        out_ref[:, :] = gather_ref[:, :, :].reshape(
            ROWS, D_MODEL).astype(jnp.float32)
        for rdma in sends_a + sends_b:
            rdma.wait_send()

    out = pl.pallas_call(
        body,
        out_shape=jax.ShapeDtypeStruct((ROWS, D_MODEL), jnp.float32),
        in_specs=[pl.BlockSpec(memory_space=pltpu.VMEM)] * 5,
        out_specs=pl.BlockSpec(memory_space=pltpu.VMEM),
        scratch_shapes=[
            pltpu.VMEM((N_DEV, ROWS_PER, D_MODEL), jnp.bfloat16),
            pltpu.VMEM((N_DEV, ROWS_PER, D_MODEL), jnp.bfloat16),
            pltpu.VMEM((ROWS_PER, D_MODEL), jnp.bfloat16),
            pltpu.VMEM((N_DEV, ROWS_PER, D_MODEL), jnp.bfloat16),
            pltpu.SemaphoreType.DMA((1,)),
            pltpu.SemaphoreType.DMA((N_DEV,)),
            pltpu.SemaphoreType.DMA((N_DEV,)),
            pltpu.SemaphoreType.DMA((N_DEV,)),
            pltpu.SemaphoreType.DMA((N_DEV,)),
        ],
        compiler_params=pltpu.CompilerParams(collective_id=0),
    )(xf, Wq, Ks, Vs, Wo)
    return out.reshape(B, SQ, D_MODEL)


# device time: 15445 ns/iter; 1.5250x vs baseline; 1.0236x over previous
import jax
import jax.numpy as jnp
from jax import lax
from jax.experimental import pallas as pl
from jax.experimental.pallas import tpu as pltpu

N_DEV = 8
B, SQ, D_MODEL = 2, 128, 512
H_PER, DH = 4, 64
ROWS = B * SQ
ROWS_PER = ROWS // N_DEV
NC = 2
CW = D_MODEL // NC


def kernel(x, Wq, K_ext, V_ext, Wo):
    my = lax.axis_index("i")
    Ks = lax.dynamic_slice_in_dim(K_ext, my * H_PER, H_PER, axis=2)
    Vs = lax.dynamic_slice_in_dim(V_ext, my * H_PER, H_PER, axis=2)
    Ks = jnp.transpose(Ks, (0, 2, 1, 3))
    Vs = jnp.transpose(Vs, (0, 2, 1, 3))
    xf = x.reshape(ROWS, D_MODEL)

    def body(x_ref, wq_ref, k_ref, v_ref, wo_ref, out_ref,
             partial_ref, comm_a, comm_b, gather_ref,
             local_sems, send_a, recv_a, send_b, recv_b):
        my_pos = lax.axis_index("i")

        barrier_sem = pltpu.get_barrier_semaphore()
        for o in range(1, N_DEV):
            t = lax.rem(my_pos + o, N_DEV)
            pl.semaphore_signal(barrier_sem, inc=1, device_id=(t,),
                                device_id_type=pl.DeviceIdType.MESH)

        q_all = jnp.dot(x_ref[:, :], wq_ref[:, :],
                        preferred_element_type=jnp.float32)
        for b in range(B):
            ctx_parts = []
            for h in range(H_PER):
                q = q_all[b * SQ:(b + 1) * SQ, h * DH:(h + 1) * DH]
                k = k_ref[b, h]
                v = v_ref[b, h]
                s = lax.dot_general(
                    q, k, (((1,), (1,)), ((), ())),
                    preferred_element_type=jnp.float32) * 0.125
                m = jnp.max(s, axis=-1, keepdims=True)
                e = jnp.exp(s - m)
                w = e / jnp.sum(e, axis=-1, keepdims=True)
                ctx_parts.append(jnp.dot(w, v,
                                         preferred_element_type=jnp.float32))
            ctx_b = jnp.concatenate(ctx_parts, axis=1)
            p_b = jnp.dot(ctx_b, wo_ref[:, :],
                          preferred_element_type=jnp.float32)
            partial_ref[4 * b:4 * (b + 1)] = p_b.astype(jnp.bfloat16).reshape(
                4, ROWS_PER, D_MODEL)

        pl.semaphore_wait(barrier_sem, N_DEV - 1)

        drains = []
        own_a = []
        for c in range(NC):
            cols = pl.ds(c * CW, CW)
            own = pltpu.make_async_copy(
                partial_ref.at[my_pos, :, cols], comm_a.at[c, 0],
                local_sems.at[c])
            own.start()
            own_a.append(own)
            for o in range(1, N_DEV):
                t = lax.rem(my_pos + o, N_DEV)
                rdma = pltpu.make_async_remote_copy(
                    src_ref=partial_ref.at[t, :, cols],
                    dst_ref=comm_a.at[c, o],
                    send_sem=send_a.at[c, o],
                    recv_sem=recv_a.at[c, o],
                    device_id=(t,),
                    device_id_type=pl.DeviceIdType.MESH,
                )
                rdma.start()
                drains.append(rdma)
        for c in range(NC):
            own_a[c].wait()
            for o in range(1, N_DEV):
                pltpu.make_async_remote_copy(
                    src_ref=comm_a.at[c, o],
                    dst_ref=comm_a.at[c, o],
                    send_sem=send_a.at[c, o],
                    recv_sem=recv_a.at[c, o],
                    device_id=(my_pos,),
                    device_id_type=pl.DeviceIdType.MESH,
                ).wait_recv()
            red = comm_a[c, 0].astype(jnp.float32)
            for o in range(1, N_DEV):
                red = red + comm_a[c, o].astype(jnp.float32)
            comm_b[c] = red.astype(jnp.bfloat16)
            for o in range(1, N_DEV):
                t = lax.rem(my_pos + o, N_DEV)
                rdma = pltpu.make_async_remote_copy(
                    src_ref=comm_b.at[c],
                    dst_ref=gather_ref.at[c, my_pos],
                    send_sem=send_b.at[c, o],
                    recv_sem=recv_b.at[c, my_pos],
                    device_id=(t,),
                    device_id_type=pl.DeviceIdType.MESH,
                )
                rdma.start()
                drains.append(rdma)
            for p in range(N_DEV):
                @pl.when(p == my_pos)
                def _():
                    gather_ref[c, p] = comm_b[c]
        for c in range(NC):
            cols = pl.ds(c * CW, CW)
            for p in range(N_DEV):
                @pl.when(p != my_pos)
                def _():
                    pltpu.make_async_remote_copy(
                        src_ref=comm_b.at[c],
                        dst_ref=gather_ref.at[c, p],
                        send_sem=send_b.at[c, p],
                        recv_sem=recv_b.at[c, p],
                        device_id=(p,),
                        device_id_type=pl.DeviceIdType.MESH,
                    ).wait_recv()
            out_ref[:, cols] = gather_ref[c].reshape(
                ROWS, CW).astype(jnp.float32)
        for rdma in drains:
            rdma.wait_send()

    out = pl.pallas_call(
        body,
        out_shape=jax.ShapeDtypeStruct((ROWS, D_MODEL), jnp.float32),
        in_specs=[pl.BlockSpec(memory_space=pltpu.VMEM)] * 5,
        out_specs=pl.BlockSpec(memory_space=pltpu.VMEM),
        scratch_shapes=[
            pltpu.VMEM((N_DEV, ROWS_PER, D_MODEL), jnp.bfloat16),
            pltpu.VMEM((NC, N_DEV, ROWS_PER, CW), jnp.bfloat16),
            pltpu.VMEM((NC, ROWS_PER, CW), jnp.bfloat16),
            pltpu.VMEM((NC, N_DEV, ROWS_PER, CW), jnp.bfloat16),
            pltpu.SemaphoreType.DMA((NC,)),
            pltpu.SemaphoreType.DMA((NC, N_DEV)),
            pltpu.SemaphoreType.DMA((NC, N_DEV)),
            pltpu.SemaphoreType.DMA((NC, N_DEV)),
            pltpu.SemaphoreType.DMA((NC, N_DEV)),
        ],
        compiler_params=pltpu.CompilerParams(collective_id=0),
    )(xf, Wq, Ks, Vs, Wo)
    return out.reshape(B, SQ, D_MODEL)
